# baseline (speedup 1.0000x reference)
import functools

import jax
import jax.numpy as jnp
from jax import lax
from jax.experimental import pallas as pl
from jax.experimental.pallas import tpu as pltpu
from jax.experimental.pallas import tpu_sc as plsc

_DIM = 256
_K = 8192
_N = 8192

_TN = 2048
_TK = 1024
_NJ = _K // _TK

_INTERPRET = False


def _vq_body(x_ref, e_ref, x2_ref, e2_ref,
             d_ref, idx_ref, enc_ref, cnt_ref, perp_ref,
             x2x_s, best_s, besti_s):
    i = pl.program_id(0)
    j = pl.program_id(1)
    ni = pl.num_programs(0)

    @pl.when(j < _NJ)
    def _distance_phase():
        @pl.when(j == 0)
        def _init():
            x2x_s[...] = 2.0 * x_ref[...]
            best_s[...] = jnp.full_like(best_s, jnp.inf)
            besti_s[...] = jnp.zeros_like(besti_s)

        e = e_ref[pl.ds(j * _TK, _TK), :]
        mm2 = lax.dot_general(
            x2x_s[...], e, (((1,), (1,)), ((), ())),
            preferred_element_type=jnp.float32,
        )
        d = (x2_ref[...] + e2_ref[0]) - mm2
        d_ref[...] = d

        rmin = jnp.min(d, axis=1, keepdims=True)
        colsf = lax.broadcasted_iota(jnp.int32, (1, _TK), 1).astype(jnp.float32)
        rargf = jnp.min(jnp.where(d == rmin, colsf, jnp.float32(2**30)),
                        axis=1, keepdims=True)
        rarg = rargf.astype(jnp.int32) + j * _TK
        upd = rmin < best_s[...]
        besti_s[...] = jnp.where(upd, rarg, besti_s[...])
        best_s[...] = jnp.where(upd, rmin, best_s[...])

        @pl.when(j == _NJ - 1)
        def _out():
            idx_ref[...] = besti_s[...]

    @pl.when(j >= _NJ)
    def _onehot_phase():
        jb = j - _NJ

        @pl.when((i == 0) & (jb == 0))
        def _init_cnt():
            cnt_ref[...] = jnp.zeros_like(cnt_ref)

        cols = lax.broadcasted_iota(jnp.int32, (_TN, _TK), 1) + jb * _TK
        enc = (besti_s[...] == cols).astype(jnp.float32)
        enc_ref[...] = enc
        cnt_ref[0, pl.ds(jb * _TK, _TK)] += jnp.sum(enc, axis=0)

        @pl.when((i == ni - 1) & (jb == _NJ - 1))
        def _finish():
            avg = cnt_ref[...] * (1.0 / _N)
            ent = jnp.sum(avg * jnp.log(avg + 1e-10))
            perp_ref[...] = jnp.exp(-ent)[None, None]


def _dist_body(x_ref, e_ref, x2_ref, e2_ref, d_ref, idx_ref,
               x2x_s, best_s, besti_s):
    j = pl.program_id(1)

    @pl.when(j == 0)
    def _init():
        x2x_s[...] = 2.0 * x_ref[...]
        best_s[...] = jnp.full_like(best_s, jnp.inf)
        besti_s[...] = jnp.zeros_like(besti_s)

    e = e_ref[pl.ds(j * _TK, _TK), :]
    mm2 = lax.dot_general(
        x2x_s[...], e, (((1,), (1,)), ((), ())),
        preferred_element_type=jnp.float32,
    )
    d = (x2_ref[...] + e2_ref[0]) - mm2
    d_ref[...] = d

    rmin = jnp.min(d, axis=1, keepdims=True)
    colsf = lax.broadcasted_iota(jnp.int32, (1, _TK), 1).astype(jnp.float32)
    rargf = jnp.min(jnp.where(d == rmin, colsf, jnp.float32(2**30)),
                    axis=1, keepdims=True)
    rarg = rargf.astype(jnp.int32) + j * _TK
    upd = rmin < best_s[...]
    besti_s[...] = jnp.where(upd, rarg, besti_s[...])
    best_s[...] = jnp.where(upd, rmin, best_s[...])

    @pl.when(j == _NJ - 1)
    def _out():
        idx_ref[...] = besti_s[...]


def _distances_argmin(flat_x, emb, x2, e2):
    return pl.pallas_call(
        _dist_body,
        grid=(_N // _TN, _NJ),
        in_specs=[
            pl.BlockSpec((_TN, _DIM), lambda i, j: (i, 0)),
            pl.BlockSpec((_K, _DIM), lambda i, j: (0, 0)),
            pl.BlockSpec((_TN, 1), lambda i, j: (i, 0)),
            pl.BlockSpec((1, 1, _TK), lambda i, j: (j, 0, 0)),
        ],
        out_specs=[
            pl.BlockSpec((_TN, _TK), lambda i, j: (i, j)),
            pl.BlockSpec((_TN, 1), lambda i, j: (i, 0)),
        ],
        out_shape=[
            jax.ShapeDtypeStruct((_N, _K), jnp.float32),
            jax.ShapeDtypeStruct((_N, 1), jnp.int32),
        ],
        scratch_shapes=[
            pltpu.VMEM((_TN, _DIM), jnp.float32),
            pltpu.VMEM((_TN, 1), jnp.float32),
            pltpu.VMEM((_TN, 1), jnp.int32),
        ],
        compiler_params=pltpu.CompilerParams(
            dimension_semantics=("arbitrary", "arbitrary"),
        ),
        interpret=_INTERPRET,
    )(flat_x, emb, x2, e2.reshape(_K // _TK, 1, _TK))


def _onehot_body(idx_ref, enc_ref, cnt_ref, perp_ref):
    i = pl.program_id(0)
    j = pl.program_id(1)
    ni = pl.num_programs(0)
    nj = pl.num_programs(1)

    @pl.when((i == 0) & (j == 0))
    def _init_cnt():
        cnt_ref[...] = jnp.zeros_like(cnt_ref)

    cols = lax.broadcasted_iota(jnp.int32, (_TN, _TK), 1) + j * _TK
    enc = (idx_ref[...] == cols).astype(jnp.float32)
    enc_ref[...] = enc
    cnt_ref[0, pl.ds(j * _TK, _TK)] += jnp.sum(enc, axis=0)

    @pl.when((i == ni - 1) & (j == nj - 1))
    def _finish():
        avg = cnt_ref[...] * (1.0 / _N)
        ent = jnp.sum(avg * jnp.log(avg + 1e-10))
        perp_ref[...] = jnp.exp(-ent)[None, None]


def _onehot_stats(idx):
    return pl.pallas_call(
        _onehot_body,
        grid=(_N // _TN, _NJ),
        in_specs=[pl.BlockSpec((_TN, 1), lambda i, j: (i, 0))],
        out_specs=[
            pl.BlockSpec((_TN, _TK), lambda i, j: (i, j)),
            pl.BlockSpec((1, _K), lambda i, j: (0, 0)),
            pl.BlockSpec((1, 1), lambda i, j: (0, 0)),
        ],
        out_shape=[
            jax.ShapeDtypeStruct((_N, _K), jnp.float32),
            jax.ShapeDtypeStruct((1, _K), jnp.float32),
            jax.ShapeDtypeStruct((1, 1), jnp.float32),
        ],
        compiler_params=pltpu.CompilerParams(
            dimension_semantics=("arbitrary", "arbitrary"),
        ),
        interpret=_INTERPRET,
    )(idx)


def _vq_main(flat_x, emb, x2, e2):
    nj = _NJ

    def dmap(i, j):
        return (i, jnp.minimum(j, nj - 1))

    def emap(i, j):
        return (i, jnp.maximum(j - nj, 0))

    return pl.pallas_call(
        _vq_body,
        grid=(_N // _TN, 2 * nj),
        in_specs=[
            pl.BlockSpec((_TN, _DIM), lambda i, j: (i, 0)),
            pl.BlockSpec((_K, _DIM), lambda i, j: (0, 0)),
            pl.BlockSpec((_TN, 1), lambda i, j: (i, 0)),
            pl.BlockSpec((1, 1, _TK), lambda i, j: (jnp.minimum(j, nj - 1), 0, 0)),
        ],
        out_specs=[
            pl.BlockSpec((_TN, _TK), dmap),
            pl.BlockSpec((_TN, 1), lambda i, j: (i, 0)),
            pl.BlockSpec((_TN, _TK), emap),
            pl.BlockSpec((1, _K), lambda i, j: (0, 0)),
            pl.BlockSpec((1, 1), lambda i, j: (0, 0)),
        ],
        out_shape=[
            jax.ShapeDtypeStruct((_N, _K), jnp.float32),
            jax.ShapeDtypeStruct((_N, 1), jnp.int32),
            jax.ShapeDtypeStruct((_N, _K), jnp.float32),
            jax.ShapeDtypeStruct((1, _K), jnp.float32),
            jax.ShapeDtypeStruct((1, 1), jnp.float32),
        ],
        scratch_shapes=[
            pltpu.VMEM((_TN, _DIM), jnp.float32),
            pltpu.VMEM((_TN, 1), jnp.float32),
            pltpu.VMEM((_TN, 1), jnp.int32),
        ],
        compiler_params=pltpu.CompilerParams(
            dimension_semantics=("arbitrary", "arbitrary"),
        ),
        interpret=_INTERPRET,
    )(flat_x, emb, x2, e2.reshape(_K // _TK, 1, _TK))


def _sc_gather(emb, idx_flat):
    info = plsc.get_sparse_core_info()
    nw = info.num_cores * info.num_subcores
    b_per_w = _N // nw
    mesh = plsc.VectorSubcoreMesh(core_axis_name="c", subcore_axis_name="s")

    @functools.partial(
        pl.kernel,
        mesh=mesh,
        out_type=jax.ShapeDtypeStruct((_N, _DIM), jnp.float32),
        scratch_types=[
            pltpu.VMEM((b_per_w,), jnp.int32),
            pltpu.VMEM((b_per_w, _DIM), jnp.float32),
            pltpu.SemaphoreType.DMA,
        ],
    )
    def gather_k(table_hbm, idx_hbm, out_hbm, idx_v, rows_v, sem):
        wid = lax.axis_index("s") * info.num_cores + lax.axis_index("c")
        base = wid * b_per_w
        pltpu.sync_copy(idx_hbm.at[pl.ds(base, b_per_w)], idx_v)
        pltpu.async_copy(table_hbm.at[idx_v], rows_v, sem).wait()
        pltpu.sync_copy(rows_v, out_hbm.at[pl.ds(base, b_per_w)])

    return gather_k(emb, idx_flat)


def kernel(inputs, embedding_weight):
    x = jnp.transpose(inputs, (0, 2, 3, 1))
    input_shape = x.shape
    flat_x = x.reshape(_N, _DIM)

    x2 = jnp.sum(flat_x**2, axis=1, keepdims=True)
    e2 = jnp.sum(embedding_weight**2, axis=1)
    distances, encoding_indices = _distances_argmin(
        flat_x, embedding_weight, x2, e2)
    quantized = _sc_gather(embedding_weight, encoding_indices.reshape(_N))
    encodings, _counts, perp = _onehot_stats(encoding_indices)

    quantized = quantized.reshape(input_shape)
    quantized_st = x + lax.stop_gradient(quantized - x)
    quantized_out = jnp.transpose(quantized_st, (0, 3, 1, 2))
    perplexity = perp.reshape(())
    return (distances, encodings, encoding_indices, quantized_out,
            quantized, perplexity)


# speedup vs baseline: 1.0084x; 1.0084x over previous
import functools

import jax
import jax.numpy as jnp
from jax import lax
from jax.experimental import pallas as pl
from jax.experimental.pallas import tpu as pltpu
from jax.experimental.pallas import tpu_sc as plsc

_DIM = 256
_K = 8192
_N = 8192

_TN = 2048
_TK = 1024
_NJ = _K // _TK

_INTERPRET = False


def _vq_body(x_ref, e_ref, x2_ref, e2_ref,
             d_ref, idx_ref, enc_ref, cnt_ref, perp_ref,
             x2x_s, best_s, besti_s):
    i = pl.program_id(0)
    j = pl.program_id(1)
    ni = pl.num_programs(0)

    @pl.when(j < _NJ)
    def _distance_phase():
        @pl.when(j == 0)
        def _init():
            x2x_s[...] = 2.0 * x_ref[...]
            best_s[...] = jnp.full_like(best_s, jnp.inf)
            besti_s[...] = jnp.zeros_like(besti_s)

        e = e_ref[pl.ds(j * _TK, _TK), :]
        mm2 = lax.dot_general(
            x2x_s[...], e, (((1,), (1,)), ((), ())),
            preferred_element_type=jnp.float32,
        )
        d = (x2_ref[...] + e2_ref[0]) - mm2
        d_ref[...] = d

        rmin = jnp.min(d, axis=1, keepdims=True)
        colsf = lax.broadcasted_iota(jnp.int32, (1, _TK), 1).astype(jnp.float32)
        rargf = jnp.min(jnp.where(d == rmin, colsf, jnp.float32(2**30)),
                        axis=1, keepdims=True)
        rarg = rargf.astype(jnp.int32) + j * _TK
        upd = rmin < best_s[...]
        besti_s[...] = jnp.where(upd, rarg, besti_s[...])
        best_s[...] = jnp.where(upd, rmin, best_s[...])

        @pl.when(j == _NJ - 1)
        def _out():
            idx_ref[...] = besti_s[...]

    @pl.when(j >= _NJ)
    def _onehot_phase():
        jb = j - _NJ

        @pl.when((i == 0) & (jb == 0))
        def _init_cnt():
            cnt_ref[...] = jnp.zeros_like(cnt_ref)

        cols = lax.broadcasted_iota(jnp.int32, (_TN, _TK), 1) + jb * _TK
        enc = (besti_s[...] == cols).astype(jnp.float32)
        enc_ref[...] = enc
        cnt_ref[0, pl.ds(jb * _TK, _TK)] += jnp.sum(enc, axis=0)

        @pl.when((i == ni - 1) & (jb == _NJ - 1))
        def _finish():
            avg = cnt_ref[...] * (1.0 / _N)
            ent = jnp.sum(avg * jnp.log(avg + 1e-10))
            perp_ref[...] = jnp.exp(-ent)[None, None]


def _vq_main(flat_x, emb, x2, e2):
    nj = _NJ

    def dmap(i, j):
        return (i, jnp.minimum(j, nj - 1))

    def emap(i, j):
        return (i, jnp.maximum(j - nj, 0))

    return pl.pallas_call(
        _vq_body,
        grid=(_N // _TN, 2 * nj),
        in_specs=[
            pl.BlockSpec((_TN, _DIM), lambda i, j: (i, 0)),
            pl.BlockSpec((_K, _DIM), lambda i, j: (0, 0)),
            pl.BlockSpec((_TN, 1), lambda i, j: (i, 0)),
            pl.BlockSpec((1, 1, _TK), lambda i, j: (jnp.minimum(j, nj - 1), 0, 0)),
        ],
        out_specs=[
            pl.BlockSpec((_TN, _TK), dmap),
            pl.BlockSpec((_TN, 1), lambda i, j: (i, 0)),
            pl.BlockSpec((_TN, _TK), emap),
            pl.BlockSpec((1, _K), lambda i, j: (0, 0)),
            pl.BlockSpec((1, 1), lambda i, j: (0, 0)),
        ],
        out_shape=[
            jax.ShapeDtypeStruct((_N, _K), jnp.float32),
            jax.ShapeDtypeStruct((_N, 1), jnp.int32),
            jax.ShapeDtypeStruct((_N, _K), jnp.float32),
            jax.ShapeDtypeStruct((1, _K), jnp.float32),
            jax.ShapeDtypeStruct((1, 1), jnp.float32),
        ],
        scratch_shapes=[
            pltpu.VMEM((_TN, _DIM), jnp.float32),
            pltpu.VMEM((_TN, 1), jnp.float32),
            pltpu.VMEM((_TN, 1), jnp.int32),
        ],
        compiler_params=pltpu.CompilerParams(
            dimension_semantics=("arbitrary", "arbitrary"),
        ),
        interpret=_INTERPRET,
    )(flat_x, emb, x2, e2.reshape(_K // _TK, 1, _TK))


def _sc_gather(emb, idx_flat):
    info = plsc.get_sparse_core_info()
    nw = info.num_cores * info.num_subcores
    b_per_w = _N // nw
    mesh = plsc.VectorSubcoreMesh(core_axis_name="c", subcore_axis_name="s")

    @functools.partial(
        pl.kernel,
        mesh=mesh,
        out_type=jax.ShapeDtypeStruct((_N, _DIM), jnp.float32),
        scratch_types=[
            pltpu.VMEM((b_per_w,), jnp.int32),
            pltpu.VMEM((b_per_w, _DIM), jnp.float32),
            pltpu.SemaphoreType.DMA,
        ],
    )
    def gather_k(table_hbm, idx_hbm, out_hbm, idx_v, rows_v, sem):
        wid = lax.axis_index("s") * info.num_cores + lax.axis_index("c")
        base = wid * b_per_w
        pltpu.sync_copy(idx_hbm.at[pl.ds(base, b_per_w)], idx_v)
        pltpu.async_copy(table_hbm.at[idx_v], rows_v, sem).wait()
        pltpu.sync_copy(rows_v, out_hbm.at[pl.ds(base, b_per_w)])

    return gather_k(emb, idx_flat)


def kernel(inputs, embedding_weight):
    x = jnp.transpose(inputs, (0, 2, 3, 1))
    input_shape = x.shape
    flat_x = x.reshape(_N, _DIM)

    x2 = jnp.sum(flat_x**2, axis=1, keepdims=True)
    e2 = jnp.sum(embedding_weight**2, axis=1)
    distances, encoding_indices, encodings, _counts, perp = _vq_main(
        flat_x, embedding_weight, x2, e2)
    quantized = _sc_gather(embedding_weight, encoding_indices.reshape(_N))

    quantized = quantized.reshape(input_shape)
    quantized_st = x + lax.stop_gradient(quantized - x)
    quantized_out = jnp.transpose(quantized_st, (0, 3, 1, 2))
    perplexity = perp.reshape(())
    return (distances, encodings, encoding_indices, quantized_out,
            quantized, perplexity)
